# baseline (speedup 1.0000x reference)
import functools

import jax
import jax.numpy as jnp
from jax import lax
from jax.experimental import pallas as pl
from jax.experimental.pallas import tpu as pltpu
from jax.experimental.pallas import tpu_sc as plsc

IDX_PER_STREAM = 128
STREAMS_IN_FLIGHT = 8
N_WORKERS = 32
PACK = 4


XPAD = 128


def _make_gather(bsz, seq, es):
    bpw = bsz // N_WORKERS
    bq_n = bpw // IDX_PER_STREAM
    t_per_iter = STREAMS_IN_FLIGHT // bq_n
    mesh = plsc.VectorSubcoreMesh(core_axis_name="c", subcore_axis_name="s")

    @functools.partial(
        pl.kernel,
        mesh=mesh,
        out_type=jax.ShapeDtypeStruct((bsz * seq, es), jnp.float32),
        scratch_types=[
            pltpu.VMEM((bpw, XPAD), jnp.int32),
            pltpu.VMEM((STREAMS_IN_FLIGHT, IDX_PER_STREAM), jnp.int32),
            pltpu.VMEM((STREAMS_IN_FLIGHT * IDX_PER_STREAM, es), jnp.float32),
            pltpu.SemaphoreType.DMA,
        ],
        compiler_params=pltpu.CompilerParams(
            use_tc_tiling_on_sc=False, needs_layout_passes=False
        ),
    )
    def gather_k(x_hbm, table_hbm, out_hbm, x_v, tidx_v, g_v, gsem):
        wid = lax.axis_index("s") * 2 + lax.axis_index("c")
        pltpu.sync_copy(x_hbm.at[pl.ds(wid * bpw, bpw)], x_v)
        iota16 = lax.iota(jnp.int32, 16)

        def outer(s, carry):
            for dt in range(t_per_iter):
                t = t_per_iter * s + dt
                tvec = jnp.zeros((16,), jnp.int32) + t
                for bq in range(bq_n):
                    for v in range(IDX_PER_STREAM // 16):
                        row0 = bq * IDX_PER_STREAM + v * 16
                        vals = plsc.load_gather(x_v, [iota16 + row0, tvec])
                        tidx_v[dt * bq_n + bq, pl.ds(v * 16, 16)] = vals
            cps = []
            for j in range(STREAMS_IN_FLIGHT):
                cp = pltpu.async_copy(
                    table_hbm.at[tidx_v.at[j]],
                    g_v.at[pl.ds(j * IDX_PER_STREAM, IDX_PER_STREAM)],
                    gsem,
                )
                cps.append(cp)
            for cp in cps:
                cp.wait()
            for dt in range(t_per_iter):
                t = t_per_iter * s + dt
                row0 = t * bsz + wid * bpw
                pltpu.sync_copy(
                    g_v.at[pl.ds(dt * bpw, bpw)],
                    out_hbm.at[pl.ds(row0, bpw)],
                )
            return carry

        lax.fori_loop(0, seq // t_per_iter, outer, 0)

    return gather_k


def _len_body(x_ref, out_ref):
    rows, pw = out_ref.shape
    cnt = jnp.sum((x_ref[...] != 0).astype(jnp.int32), axis=1, keepdims=True)
    cnt4 = cnt.reshape(rows, PACK)
    parts = [
        jnp.broadcast_to(cnt4[:, q : q + 1], (rows, pw // PACK))
        for q in range(PACK)
    ]
    out_ref[...] = jnp.concatenate(parts, axis=1)


def _gru_body(len_ref, e_ref, wih_ref, whh_ref, b_ref, bhn_ref, out_ref,
              h_scr):
    t = pl.program_id(0)
    n_steps = pl.num_programs(0)
    pw = h_scr.shape[1]

    @pl.when(t == 0)
    def _init():
        h_scr[...] = jnp.zeros_like(h_scr)

    h4 = h_scr[...]
    e_t = e_ref[0]
    ge = jnp.dot(e_t.astype(jnp.bfloat16), wih_ref[...],
                 preferred_element_type=jnp.float32)
    gh = jnp.dot(h4.astype(jnp.bfloat16), whh_ref[...],
                 preferred_element_type=jnp.float32)
    g = ge + b_ref[...]
    rz = jax.nn.sigmoid(g[:, : 2 * pw] + gh[:, : 2 * pw])
    r = rz[:, :pw]
    z = rz[:, pw:]
    n = jnp.tanh(g[:, 2 * pw :] + r * (gh[:, 2 * pw :] + bhn_ref[...]))
    h_new = n + z * (h4 - n)
    keep = t < len_ref[...]
    h_scr[...] = jnp.where(keep, h_new, h4)

    @pl.when(t == n_steps - 1)
    def _fin():
        out_ref[...] = h_scr[...]


def kernel(x, emb, W_ih, W_hh, b_ih, b_hh):
    x = x.astype(jnp.int32)
    bsz, seq = x.shape
    es = emb.shape[1]
    hs = W_hh.shape[1]
    rows = bsz // PACK

    xp = jnp.pad(x, ((0, 0), (0, XPAD - seq)))
    gather = _make_gather(bsz, seq, es)
    e4 = gather(xp, emb).reshape(seq, rows, PACK * es)

    WihT = W_ih.T
    WhhT = W_hh.T
    eye = jnp.eye(PACK, dtype=jnp.float32)

    def gate_major(w):
        return jnp.concatenate(
            [jnp.kron(eye, w[:, i * hs : (i + 1) * hs]) for i in range(3)],
            axis=1,
        )

    WihBD = gate_major(WihT).astype(jnp.bfloat16)
    WhhBD = gate_major(WhhT).astype(jnp.bfloat16)
    b4 = jnp.concatenate(
        [jnp.tile(b_ih[i * hs : (i + 1) * hs]
                  + (b_hh[i * hs : (i + 1) * hs] if i < 2 else 0.0), PACK)
         for i in range(3)]
    ).reshape(1, 3 * PACK * hs)
    bhn = jnp.tile(b_hh[2 * hs :], PACK).reshape(1, PACK * hs)

    len4 = pl.pallas_call(
        _len_body,
        in_specs=[pl.BlockSpec((bsz, seq), lambda: (0, 0))],
        out_specs=pl.BlockSpec((rows, PACK * hs), lambda: (0, 0)),
        out_shape=jax.ShapeDtypeStruct((rows, PACK * hs), jnp.int32),
    )(x)

    h4 = pl.pallas_call(
        _gru_body,
        grid=(seq,),
        in_specs=[
            pl.BlockSpec((rows, PACK * hs), lambda t: (0, 0)),
            pl.BlockSpec((1, rows, PACK * es), lambda t: (t, 0, 0)),
            pl.BlockSpec((PACK * es, 3 * PACK * hs), lambda t: (0, 0)),
            pl.BlockSpec((PACK * hs, 3 * PACK * hs), lambda t: (0, 0)),
            pl.BlockSpec((1, 3 * PACK * hs), lambda t: (0, 0)),
            pl.BlockSpec((1, PACK * hs), lambda t: (0, 0)),
        ],
        out_specs=pl.BlockSpec((rows, PACK * hs), lambda t: (0, 0)),
        out_shape=jax.ShapeDtypeStruct((rows, PACK * hs), jnp.float32),
        scratch_shapes=[
            pltpu.VMEM((rows, PACK * hs), jnp.float32),
        ],
        compiler_params=pltpu.CompilerParams(
            dimension_semantics=("arbitrary",)
        ),
    )(len4, e4, WihBD, WhhBD, b4, bhn)
    return h4.reshape(rows, PACK, hs).reshape(bsz, hs)


# speedup vs baseline: 1.0492x; 1.0492x over previous
import functools

import jax
import jax.numpy as jnp
from jax import lax
from jax.experimental import pallas as pl
from jax.experimental.pallas import tpu as pltpu
from jax.experimental.pallas import tpu_sc as plsc

IDX_PER_STREAM = 128
STREAMS_IN_FLIGHT = 8
N_WORKERS = 32
PACK = 4


XPAD = 128


def _make_gather(bsz, seq, es, t_lo, t_hi):
    bpw = bsz // N_WORKERS
    bq_n = bpw // IDX_PER_STREAM
    t_per_iter = STREAMS_IN_FLIGHT // bq_n
    mesh = plsc.VectorSubcoreMesh(core_axis_name="c", subcore_axis_name="s")

    @functools.partial(
        pl.kernel,
        mesh=mesh,
        out_type=jax.ShapeDtypeStruct((bsz * (t_hi - t_lo), es), jnp.float32),
        scratch_types=[
            pltpu.VMEM((bpw, XPAD), jnp.int32),
            pltpu.VMEM((STREAMS_IN_FLIGHT, IDX_PER_STREAM), jnp.int32),
            pltpu.VMEM((STREAMS_IN_FLIGHT * IDX_PER_STREAM, es), jnp.float32),
            pltpu.SemaphoreType.DMA,
        ],
        compiler_params=pltpu.CompilerParams(
            use_tc_tiling_on_sc=False, needs_layout_passes=False
        ),
    )
    def gather_k(x_hbm, table_hbm, out_hbm, x_v, tidx_v, g_v, gsem):
        wid = lax.axis_index("s") * 2 + lax.axis_index("c")
        pltpu.sync_copy(x_hbm.at[pl.ds(wid * bpw, bpw)], x_v)
        iota16 = lax.iota(jnp.int32, 16)

        def outer(s, carry):
            for dt in range(t_per_iter):
                t = t_lo + t_per_iter * s + dt
                tvec = jnp.zeros((16,), jnp.int32) + t
                for bq in range(bq_n):
                    for v in range(IDX_PER_STREAM // 16):
                        row0 = bq * IDX_PER_STREAM + v * 16
                        vals = plsc.load_gather(x_v, [iota16 + row0, tvec])
                        tidx_v[dt * bq_n + bq, pl.ds(v * 16, 16)] = vals
            cps = []
            for j in range(STREAMS_IN_FLIGHT):
                cp = pltpu.async_copy(
                    table_hbm.at[tidx_v.at[j]],
                    g_v.at[pl.ds(j * IDX_PER_STREAM, IDX_PER_STREAM)],
                    gsem,
                )
                cps.append(cp)
            for cp in cps:
                cp.wait()
            for dt in range(t_per_iter):
                t = t_per_iter * s + dt
                row0 = t * bsz + wid * bpw
                pltpu.sync_copy(
                    g_v.at[pl.ds(dt * bpw, bpw)],
                    out_hbm.at[pl.ds(row0, bpw)],
                )
            return carry

        lax.fori_loop(0, (t_hi - t_lo) // t_per_iter, outer, 0)

    return gather_k


def _len_body(x_ref, out_ref):
    rows, pw = out_ref.shape
    cnt = jnp.sum((x_ref[...] != 0).astype(jnp.int32), axis=1, keepdims=True)
    cnt4 = cnt.reshape(rows, PACK)
    parts = [
        jnp.broadcast_to(cnt4[:, q : q + 1], (rows, pw // PACK))
        for q in range(PACK)
    ]
    out_ref[...] = jnp.concatenate(parts, axis=1)


def _make_gru_body(t_lo):
    def _gru_body(len_ref, e_ref, h0_ref, wih_ref, whh_ref, b_ref, bhn_ref,
                  out_ref, h_scr):
        t = pl.program_id(0)
        n_steps = pl.num_programs(0)
        pw = h_scr.shape[1]

        @pl.when(t == 0)
        def _init():
            h_scr[...] = h0_ref[...]

        h4 = h_scr[...]
        e_t = e_ref[0]
        ge = jnp.dot(e_t.astype(jnp.bfloat16), wih_ref[...],
                     preferred_element_type=jnp.float32)
        gh = jnp.dot(h4.astype(jnp.bfloat16), whh_ref[...],
                     preferred_element_type=jnp.float32)
        g = ge + b_ref[...]
        rz = jax.nn.sigmoid(g[:, : 2 * pw] + gh[:, : 2 * pw])
        r = rz[:, :pw]
        z = rz[:, pw:]
        n = jnp.tanh(g[:, 2 * pw :] + r * (gh[:, 2 * pw :] + bhn_ref[...]))
        h_new = n + z * (h4 - n)
        keep = (t + t_lo) < len_ref[...]
        h_scr[...] = jnp.where(keep, h_new, h4)

        @pl.when(t == n_steps - 1)
        def _fin():
            out_ref[...] = h_scr[...]

    return _gru_body


def kernel(x, emb, W_ih, W_hh, b_ih, b_hh):
    x = x.astype(jnp.int32)
    bsz, seq = x.shape
    es = emb.shape[1]
    hs = W_hh.shape[1]
    rows = bsz // PACK

    xp = jnp.pad(x, ((0, 0), (0, XPAD - seq)))

    WihT = W_ih.T
    WhhT = W_hh.T
    eye = jnp.eye(PACK, dtype=jnp.float32)

    def gate_major(w):
        return jnp.concatenate(
            [jnp.kron(eye, w[:, i * hs : (i + 1) * hs]) for i in range(3)],
            axis=1,
        )

    WihBD = gate_major(WihT).astype(jnp.bfloat16)
    WhhBD = gate_major(WhhT).astype(jnp.bfloat16)
    b4 = jnp.concatenate(
        [jnp.tile(b_ih[i * hs : (i + 1) * hs]
                  + (b_hh[i * hs : (i + 1) * hs] if i < 2 else 0.0), PACK)
         for i in range(3)]
    ).reshape(1, 3 * PACK * hs)
    bhn = jnp.tile(b_hh[2 * hs :], PACK).reshape(1, PACK * hs)

    len4 = pl.pallas_call(
        _len_body,
        in_specs=[pl.BlockSpec((bsz, seq), lambda: (0, 0))],
        out_specs=pl.BlockSpec((rows, PACK * hs), lambda: (0, 0)),
        out_shape=jax.ShapeDtypeStruct((rows, PACK * hs), jnp.int32),
    )(x)

    n_split = 5
    tsp = seq // n_split
    h4 = jnp.zeros((rows, PACK * hs), jnp.float32)
    for i in range(n_split):
        t_lo = i * tsp
        gather = _make_gather(bsz, seq, es, t_lo, t_lo + tsp)
        e4 = gather(xp, emb).reshape(tsp, rows, PACK * es)
        h4 = pl.pallas_call(
            _make_gru_body(t_lo),
            grid=(tsp,),
            in_specs=[
                pl.BlockSpec((rows, PACK * hs), lambda t: (0, 0)),
                pl.BlockSpec((1, rows, PACK * es), lambda t: (t, 0, 0)),
                pl.BlockSpec((rows, PACK * hs), lambda t: (0, 0)),
                pl.BlockSpec((PACK * es, 3 * PACK * hs), lambda t: (0, 0)),
                pl.BlockSpec((PACK * hs, 3 * PACK * hs), lambda t: (0, 0)),
                pl.BlockSpec((1, 3 * PACK * hs), lambda t: (0, 0)),
                pl.BlockSpec((1, PACK * hs), lambda t: (0, 0)),
            ],
            out_specs=pl.BlockSpec((rows, PACK * hs), lambda t: (0, 0)),
            out_shape=jax.ShapeDtypeStruct((rows, PACK * hs), jnp.float32),
            scratch_shapes=[
                pltpu.VMEM((rows, PACK * hs), jnp.float32),
            ],
            compiler_params=pltpu.CompilerParams(
                dimension_semantics=("arbitrary",)
            ),
        )(len4, e4, h4, WihBD, WhhBD, b4, bhn)
    return h4.reshape(rows, PACK, hs).reshape(bsz, hs)


# speedup vs baseline: 1.0515x; 1.0022x over previous
import functools

import jax
import jax.numpy as jnp
from jax import lax
from jax.experimental import pallas as pl
from jax.experimental.pallas import tpu as pltpu
from jax.experimental.pallas import tpu_sc as plsc

IDX_PER_STREAM = 128
STREAMS_IN_FLIGHT = 8
N_WORKERS = 32
PACK = 4


XPAD = 128


def _make_gather(bsz, seq, es, t_lo, t_hi):
    bpw = bsz // N_WORKERS
    bq_n = bpw // IDX_PER_STREAM
    t_per_iter = STREAMS_IN_FLIGHT // bq_n
    mesh = plsc.VectorSubcoreMesh(core_axis_name="c", subcore_axis_name="s")

    @functools.partial(
        pl.kernel,
        mesh=mesh,
        out_type=jax.ShapeDtypeStruct((bsz * (t_hi - t_lo), es), jnp.float32),
        scratch_types=[
            pltpu.VMEM((bpw, XPAD), jnp.int32),
            pltpu.VMEM((STREAMS_IN_FLIGHT, IDX_PER_STREAM), jnp.int32),
            pltpu.VMEM((STREAMS_IN_FLIGHT * IDX_PER_STREAM, es), jnp.float32),
            pltpu.SemaphoreType.DMA,
        ],
        compiler_params=pltpu.CompilerParams(
            use_tc_tiling_on_sc=False, needs_layout_passes=False
        ),
    )
    def gather_k(x_hbm, table_hbm, out_hbm, x_v, tidx_v, g_v, gsem):
        wid = lax.axis_index("s") * 2 + lax.axis_index("c")
        pltpu.sync_copy(x_hbm.at[pl.ds(wid * bpw, bpw)], x_v)
        iota16 = lax.iota(jnp.int32, 16)

        def outer(s, carry):
            for dt in range(t_per_iter):
                t = t_lo + t_per_iter * s + dt
                tvec = jnp.zeros((16,), jnp.int32) + t
                for bq in range(bq_n):
                    for v in range(IDX_PER_STREAM // 16):
                        row0 = bq * IDX_PER_STREAM + v * 16
                        vals = plsc.load_gather(x_v, [iota16 + row0, tvec])
                        tidx_v[dt * bq_n + bq, pl.ds(v * 16, 16)] = vals
            cps = []
            for j in range(STREAMS_IN_FLIGHT):
                cp = pltpu.async_copy(
                    table_hbm.at[tidx_v.at[j]],
                    g_v.at[pl.ds(j * IDX_PER_STREAM, IDX_PER_STREAM)],
                    gsem,
                )
                cps.append(cp)
            for cp in cps:
                cp.wait()
            for dt in range(t_per_iter):
                t = t_per_iter * s + dt
                row0 = t * bsz + wid * bpw
                pltpu.sync_copy(
                    g_v.at[pl.ds(dt * bpw, bpw)],
                    out_hbm.at[pl.ds(row0, bpw)],
                )
            return carry

        lax.fori_loop(0, (t_hi - t_lo) // t_per_iter, outer, 0)

    return gather_k


def _len_body(x_ref, out_ref):
    rows, pw = out_ref.shape
    cnt = jnp.sum((x_ref[...] != 0).astype(jnp.int32), axis=1, keepdims=True)
    cnt4 = cnt.reshape(rows, PACK)
    parts = [
        jnp.broadcast_to(cnt4[:, q : q + 1], (rows, pw // PACK))
        for q in range(PACK)
    ]
    out_ref[...] = jnp.concatenate(parts, axis=1)


def _make_gru_body(t_lo):
    def _gru_body(len_ref, e_ref, h0_ref, wih_ref, whh_ref, b_ref, bhn_ref,
                  out_ref, h_scr):
        t = pl.program_id(0)
        n_steps = pl.num_programs(0)
        pw = h_scr.shape[1]

        @pl.when(t == 0)
        def _init():
            h_scr[...] = h0_ref[...]

        h4 = h_scr[...]
        e_t = e_ref[0]
        ge = jnp.dot(e_t.astype(jnp.bfloat16), wih_ref[...],
                     preferred_element_type=jnp.float32)
        gh = jnp.dot(h4.astype(jnp.bfloat16), whh_ref[...],
                     preferred_element_type=jnp.float32)
        g = ge + b_ref[...]
        rz = jax.nn.sigmoid(g[:, : 2 * pw] + gh[:, : 2 * pw])
        r = rz[:, :pw]
        z = rz[:, pw:]
        n = jnp.tanh(g[:, 2 * pw :] + r * (gh[:, 2 * pw :] + bhn_ref[...]))
        h_new = n + z * (h4 - n)
        keep = (t + t_lo) < len_ref[...]
        h_scr[...] = jnp.where(keep, h_new, h4)

        @pl.when(t == n_steps - 1)
        def _fin():
            out_ref[...] = h_scr[...]

    return _gru_body


def kernel(x, emb, W_ih, W_hh, b_ih, b_hh):
    x = x.astype(jnp.int32)
    bsz, seq = x.shape
    es = emb.shape[1]
    hs = W_hh.shape[1]
    rows = bsz // PACK

    xp = jnp.pad(x, ((0, 0), (0, XPAD - seq)))

    WihT = W_ih.T
    WhhT = W_hh.T
    eye = jnp.eye(PACK, dtype=jnp.float32)

    def gate_major(w):
        return jnp.concatenate(
            [jnp.kron(eye, w[:, i * hs : (i + 1) * hs]) for i in range(3)],
            axis=1,
        )

    WihBD = gate_major(WihT).astype(jnp.bfloat16)
    WhhBD = gate_major(WhhT).astype(jnp.bfloat16)
    b4 = jnp.concatenate(
        [jnp.tile(b_ih[i * hs : (i + 1) * hs]
                  + (b_hh[i * hs : (i + 1) * hs] if i < 2 else 0.0), PACK)
         for i in range(3)]
    ).reshape(1, 3 * PACK * hs)
    bhn = jnp.tile(b_hh[2 * hs :], PACK).reshape(1, PACK * hs)

    len4 = pl.pallas_call(
        _len_body,
        in_specs=[pl.BlockSpec((bsz, seq), lambda: (0, 0))],
        out_specs=pl.BlockSpec((rows, PACK * hs), lambda: (0, 0)),
        out_shape=jax.ShapeDtypeStruct((rows, PACK * hs), jnp.int32),
    )(x)

    n_split = 2
    tsp = seq // n_split
    h4 = jnp.zeros((rows, PACK * hs), jnp.float32)
    for i in range(n_split):
        t_lo = i * tsp
        gather = _make_gather(bsz, seq, es, t_lo, t_lo + tsp)
        e4 = gather(xp, emb).reshape(tsp, rows, PACK * es)
        h4 = pl.pallas_call(
            _make_gru_body(t_lo),
            grid=(tsp,),
            in_specs=[
                pl.BlockSpec((rows, PACK * hs), lambda t: (0, 0)),
                pl.BlockSpec((1, rows, PACK * es), lambda t: (t, 0, 0)),
                pl.BlockSpec((rows, PACK * hs), lambda t: (0, 0)),
                pl.BlockSpec((PACK * es, 3 * PACK * hs), lambda t: (0, 0)),
                pl.BlockSpec((PACK * hs, 3 * PACK * hs), lambda t: (0, 0)),
                pl.BlockSpec((1, 3 * PACK * hs), lambda t: (0, 0)),
                pl.BlockSpec((1, PACK * hs), lambda t: (0, 0)),
            ],
            out_specs=pl.BlockSpec((rows, PACK * hs), lambda t: (0, 0)),
            out_shape=jax.ShapeDtypeStruct((rows, PACK * hs), jnp.float32),
            scratch_shapes=[
                pltpu.VMEM((rows, PACK * hs), jnp.float32),
            ],
            compiler_params=pltpu.CompilerParams(
                dimension_semantics=("arbitrary",)
            ),
        )(len4, e4, h4, WihBD, WhhBD, b4, bhn)
    return h4.reshape(rows, PACK, hs).reshape(bsz, hs)
